# baseline (speedup 1.0000x reference)
import functools

import jax
import jax.numpy as jnp
from jax import lax
from jax.experimental import pallas as pl
from jax.experimental.pallas import tpu as pltpu
from jax.experimental.pallas import tpu_sc as plsc

_T = 4096
_L = 64
_D = 256
_H = _T // 2
_MT = 128
_NT = _H // _MT
_NW = 16
_CK = _T // _NW


def _crf_tc_body(feats_ref, w_ref, b_row_ref, start_ref,
                 t_ref, tt_ref, end_col_ref,
                 score_ref, last_ref, bpsa_ref, bpsb_ref,
                 em_ref, srows_ref, emodd_ref, st_ref, tb_ref, mxat_ref,
                 bpat_ref, bpbt_ref):
    f32 = jnp.float32
    i32 = jnp.int32

    em_ref[...] = (
        jnp.dot(feats_ref[...], w_ref[...], preferred_element_type=f32)
        + b_row_ref[...]
    )

    siota_c = lax.broadcasted_iota(i32, (_L, 1), 0).astype(f32)

    tmat = t_ref[...]
    ttmat = tt_ref[...]

    s0 = start_ref[...] + em_ref[0:1, :]

    def fwd_pair(it, s_row):
        srows_ref[pl.ds(it, 1), :] = s_row
        swt = s_row + tmat
        mxa = jnp.max(swt, axis=1, keepdims=True)
        em_row = em_ref[pl.ds(2 * it + 1, 1), :]
        emodd_ref[pl.ds(it, 1), :] = em_row
        em_colb = jnp.broadcast_to(em_row.reshape(_L, 1), (_L, _L))
        s_colb = em_colb + mxa
        swt2 = s_colb + ttmat
        mxb = jnp.max(swt2, axis=0, keepdims=True)
        return em_ref[pl.ds(2 * it + 2, 1), :] + mxb

    s_row = lax.fori_loop(0, _H - 1, fwd_pair, s0)
    srows_ref[_H - 1:_H, :] = s_row
    emodd_ref[_H - 1:_H, :] = em_ref[_T - 1:_T, :]

    swt = s_row + tmat
    mxa = jnp.max(swt, axis=1, keepdims=True)
    em_col = em_ref[_T - 1:_T, :].reshape(_L, 1)
    final = (em_col + mxa) + end_col_ref[...]
    vs = jnp.max(final)
    score_ref[...] = jnp.full((1, 1), vs, f32)
    last = jnp.min(jnp.where(final == vs, siota_c, float(_L)),
                   axis=0, keepdims=True).astype(i32)
    last_ref[...] = jnp.broadcast_to(last, (1, 16))

    for j in range(_L):
        tb_ref[64 * j:64 * j + 64, :] = jnp.broadcast_to(
            t_ref[:, j:j + 1], (_L, _MT))

    def argmax_pass(state_t_ref, out_idx_ref, out_mx_ref):
        def tile_body(t, _):
            base = pl.multiple_of(t * _MT, _MT)
            acc = jnp.full((_L, _MT), -jnp.inf, f32)
            idx = jnp.zeros((_L, _MT), i32)
            for j in range(_L):
                c = tb_ref[64 * j:64 * j + 64, :] + \
                    state_t_ref[j:j + 1, pl.ds(base, _MT)]
                gt = c > acc
                acc = jnp.where(gt, c, acc)
                idx = jnp.where(gt, j, idx)
            out_idx_ref[:, pl.ds(base, _MT)] = idx
            if out_mx_ref is not None:
                out_mx_ref[:, pl.ds(base, _MT)] = acc
            return 0
        lax.fori_loop(0, _NT, tile_body, 0)

    st_ref[...] = srows_ref[...].T
    argmax_pass(st_ref, bpat_ref, mxat_ref)
    st_ref[...] = emodd_ref[...].T + mxat_ref[...]
    argmax_pass(st_ref, bpbt_ref, None)

    bpsa_ref[...] = bpat_ref[...].T
    bpsb_ref[...] = bpbt_ref[...].T
    bpsb_ref[_H - 1:_H, :] = lax.broadcasted_iota(i32, (1, _L), 1)


_sc_mesh = plsc.VectorSubcoreMesh(core_axis_name="c", subcore_axis_name="s",
                                  num_cores=1)


@functools.partial(
    pl.kernel, mesh=_sc_mesh,
    compiler_params=pltpu.CompilerParams(needs_layout_passes=False),
    out_type=jax.ShapeDtypeStruct((_T,), jnp.int32),
    scratch_types=[
        pltpu.VMEM((_CK * _L,), jnp.int32),
        pltpu.VMEM((_CK * _L,), jnp.int32),
        pltpu.VMEM((_L,), jnp.int32),
        pltpu.VMEM((16,), jnp.int32),
        pltpu.VMEM((_NW * _L,), jnp.int32),
        pltpu.VMEM((_NW * 16,), jnp.int32),
        pltpu.VMEM((_CK,), jnp.int32),
        pltpu.VMEM_SHARED((_NW * _L,), jnp.int32),
        pltpu.VMEM_SHARED((_NW * 16,), jnp.int32),
    ],
)
def _sc_backtrace(bps_hbm, last_hbm, out_hbm,
                  chunk_v, seg_v, exit_v, lastent_v, exall_v, entries_v,
                  path_v, sh_exits, sh_entries):
    i32 = jnp.int32
    wid = lax.axis_index("s")
    base = wid * _CK

    pltpu.sync_copy(bps_hbm.at[pl.ds(base * _L, _CK * _L)], chunk_v)
    pltpu.sync_copy(last_hbm, lastent_v)

    lane = lax.broadcasted_iota(i32, (16,), 0)

    for g in range(_L // 16):
        def body(i, ptr):
            r = _CK - 1 - i
            p = plsc.load_gather(chunk_v, [r * _L + ptr])
            seg_v[pl.ds(r * _L + 16 * g, 16)] = p
            return p
        pfin = lax.fori_loop(0, _CK, body, lane + 16 * g)
        exit_v[pl.ds(16 * g, 16)] = pfin

    pltpu.sync_copy(exit_v, sh_exits.at[pl.ds(wid * _L, _L)])
    plsc.subcore_barrier()

    @pl.when(wid == 0)
    def _():
        pltpu.sync_copy(sh_exits, exall_v)
        c0 = lastent_v[...]

        def chase(i, c):
            w = _NW - 1 - i
            entries_v[pl.ds(w * 16, 16)] = c
            return plsc.load_gather(exall_v, [w * _L + c])

        lax.fori_loop(0, _NW, chase, c0)
        pltpu.sync_copy(entries_v, sh_entries)

    plsc.subcore_barrier()

    pltpu.sync_copy(sh_entries.at[pl.ds(wid * 16, 16)], lastent_v)
    evec = lastent_v[...]
    for g in range(_CK // 16):
        rvec = lane + 16 * g
        pv = plsc.load_gather(seg_v, [rvec * _L + evec])
        path_v[pl.ds(16 * g, 16)] = pv
    pltpu.sync_copy(path_v, out_hbm.at[pl.ds(base, _CK)])


def kernel(feats, batch_sizes, W, b, start_transition, transitions,
           end_transition):
    del batch_sizes
    score, last16, bpsa, bpsb = pl.pallas_call(
        _crf_tc_body,
        out_shape=[
            jax.ShapeDtypeStruct((1, 1), jnp.float32),
            jax.ShapeDtypeStruct((1, 16), jnp.int32),
            jax.ShapeDtypeStruct((_H, _L), jnp.int32),
            jax.ShapeDtypeStruct((_H, _L), jnp.int32),
        ],
        scratch_shapes=[
            pltpu.VMEM((_T, _L), jnp.float32),
            pltpu.VMEM((_H, _L), jnp.float32),
            pltpu.VMEM((_H, _L), jnp.float32),
            pltpu.VMEM((_L, _H), jnp.float32),
            pltpu.VMEM((_L * _L, _MT), jnp.float32),
            pltpu.VMEM((_L, _H), jnp.float32),
            pltpu.VMEM((_L, _H), jnp.int32),
            pltpu.VMEM((_L, _H), jnp.int32),
        ],
    )(
        feats,
        W,
        b.reshape(1, _L),
        start_transition.reshape(1, _L),
        transitions,
        transitions.T,
        end_transition.reshape(_L, 1),
    )
    bps_full = jnp.stack([bpsa, bpsb], axis=1).reshape(_T * _L)
    path = _sc_backtrace(bps_full, last16.reshape(16))
    return score[0, 0], path


# speedup vs baseline: 1.0113x; 1.0113x over previous
import functools

import jax
import jax.numpy as jnp
from jax import lax
from jax.experimental import pallas as pl
from jax.experimental.pallas import tpu as pltpu
from jax.experimental.pallas import tpu_sc as plsc

_T = 4096
_L = 64
_D = 256
_H = _T // 2
_MT = 128
_NT = _H // _MT
_NW = 16
_CK = _T // _NW


def _crf_tc_body(feats_ref, w_ref, b_row_ref, start_ref,
                 t_ref, tt_ref, end_col_ref,
                 score_ref, last_ref, bpsa_ref, bpsb_ref,
                 em_ref, srows_ref, emodd_ref, st_ref, tb_ref, mxat_ref,
                 bpat_ref, bpbt_ref):
    f32 = jnp.float32
    i32 = jnp.int32

    em_ref[...] = (
        jnp.dot(feats_ref[...], w_ref[...], preferred_element_type=f32)
        + b_row_ref[...]
    )

    siota_c = lax.broadcasted_iota(i32, (_L, 1), 0).astype(f32)

    tmat = t_ref[...]
    ttmat = tt_ref[...]

    s0 = start_ref[...] + em_ref[0:1, :]

    def fwd_pair(it, s_row):
        srows_ref[pl.ds(it, 1), :] = s_row
        swt = s_row + tmat
        mxa = jnp.max(swt, axis=1, keepdims=True)
        em_row = em_ref[pl.ds(2 * it + 1, 1), :]
        emodd_ref[pl.ds(it, 1), :] = em_row
        em_colb = jnp.broadcast_to(em_row.reshape(_L, 1), (_L, _L))
        s_colb = em_colb + mxa
        swt2 = s_colb + ttmat
        mxb = jnp.max(swt2, axis=0, keepdims=True)
        return em_ref[pl.ds(2 * it + 2, 1), :] + mxb

    s_row = lax.fori_loop(0, _H - 1, fwd_pair, s0)
    srows_ref[_H - 1:_H, :] = s_row
    emodd_ref[_H - 1:_H, :] = em_ref[_T - 1:_T, :]

    swt = s_row + tmat
    mxa = jnp.max(swt, axis=1, keepdims=True)
    em_col = em_ref[_T - 1:_T, :].reshape(_L, 1)
    final = (em_col + mxa) + end_col_ref[...]
    vs = jnp.max(final)
    score_ref[...] = jnp.full((1, 1), vs, f32)
    last = jnp.min(jnp.where(final == vs, siota_c, float(_L)),
                   axis=0, keepdims=True).astype(i32)
    last_ref[...] = jnp.broadcast_to(last, (1, 16))

    for j in range(_L):
        tb_ref[64 * j:64 * j + 64, :] = jnp.broadcast_to(
            t_ref[:, j:j + 1], (_L, _MT))

    def argmax_pass(state_t_ref, out_idx_ref, out_mx_ref):
        def tile_body(t, _):
            base = pl.multiple_of(t * _MT, _MT)
            acc = jnp.full((_L, _MT), -jnp.inf, f32)
            idx = jnp.zeros((_L, _MT), i32)
            for j in range(_L):
                c = tb_ref[64 * j:64 * j + 64, :] + \
                    state_t_ref[j:j + 1, pl.ds(base, _MT)]
                gt = c > acc
                acc = jnp.where(gt, c, acc)
                idx = jnp.where(gt, j, idx)
            out_idx_ref[:, pl.ds(base, _MT)] = idx
            if out_mx_ref is not None:
                out_mx_ref[:, pl.ds(base, _MT)] = acc
            return 0
        lax.fori_loop(0, _NT, tile_body, 0)

    st_ref[...] = srows_ref[...].T
    argmax_pass(st_ref, bpat_ref, mxat_ref)
    st_ref[...] = emodd_ref[...].T + mxat_ref[...]
    argmax_pass(st_ref, bpbt_ref, None)

    bpsa_ref[...] = bpat_ref[...].T
    bpsb_ref[...] = bpbt_ref[...].T
    bpsb_ref[_H - 1:_H, :] = lax.broadcasted_iota(i32, (1, _L), 1)


_sc_mesh = plsc.VectorSubcoreMesh(core_axis_name="c", subcore_axis_name="s",
                                  num_cores=1)


@functools.partial(
    pl.kernel, mesh=_sc_mesh,
    compiler_params=pltpu.CompilerParams(needs_layout_passes=False),
    out_type=jax.ShapeDtypeStruct((_T,), jnp.int32),
    scratch_types=[
        pltpu.VMEM((_CK * _L,), jnp.int32),
        pltpu.VMEM((_CK * _L,), jnp.int32),
        pltpu.VMEM((_L,), jnp.int32),
        pltpu.VMEM((16,), jnp.int32),
        pltpu.VMEM((_NW * _L,), jnp.int32),
        pltpu.VMEM((_NW * 16,), jnp.int32),
        pltpu.VMEM((_CK,), jnp.int32),
        pltpu.VMEM_SHARED((_NW * _L,), jnp.int32),
        pltpu.VMEM_SHARED((_NW * 16,), jnp.int32),
    ],
)
def _sc_backtrace(bps_hbm, last_hbm, out_hbm,
                  chunk_v, seg_v, exit_v, lastent_v, exall_v, entries_v,
                  path_v, sh_exits, sh_entries):
    i32 = jnp.int32
    wid = lax.axis_index("s")
    base = wid * _CK

    pltpu.sync_copy(bps_hbm.at[pl.ds(base * _L, _CK * _L)], chunk_v)
    pltpu.sync_copy(last_hbm, lastent_v)

    lane = lax.broadcasted_iota(i32, (16,), 0)

    def body(i, ptrs):
        r = _CK - 1 - i
        new_ptrs = []
        for g in range(_L // 16):
            p = plsc.load_gather(chunk_v, [r * _L + ptrs[g]])
            seg_v[pl.ds(r * _L + 16 * g, 16)] = p
            new_ptrs.append(p)
        return tuple(new_ptrs)

    pfin = lax.fori_loop(0, _CK, body,
                         tuple(lane + 16 * g for g in range(_L // 16)))
    for g in range(_L // 16):
        exit_v[pl.ds(16 * g, 16)] = pfin[g]

    pltpu.sync_copy(exit_v, sh_exits.at[pl.ds(wid * _L, _L)])
    plsc.subcore_barrier()

    @pl.when(wid == 0)
    def _():
        pltpu.sync_copy(sh_exits, exall_v)
        c0 = lastent_v[...]

        def chase(i, c):
            w = _NW - 1 - i
            entries_v[pl.ds(w * 16, 16)] = c
            return plsc.load_gather(exall_v, [w * _L + c])

        lax.fori_loop(0, _NW, chase, c0)
        pltpu.sync_copy(entries_v, sh_entries)

    plsc.subcore_barrier()

    pltpu.sync_copy(sh_entries.at[pl.ds(wid * 16, 16)], lastent_v)
    evec = lastent_v[...]
    for g in range(_CK // 16):
        rvec = lane + 16 * g
        pv = plsc.load_gather(seg_v, [rvec * _L + evec])
        path_v[pl.ds(16 * g, 16)] = pv
    pltpu.sync_copy(path_v, out_hbm.at[pl.ds(base, _CK)])


def kernel(feats, batch_sizes, W, b, start_transition, transitions,
           end_transition):
    del batch_sizes
    score, last16, bpsa, bpsb = pl.pallas_call(
        _crf_tc_body,
        out_shape=[
            jax.ShapeDtypeStruct((1, 1), jnp.float32),
            jax.ShapeDtypeStruct((1, 16), jnp.int32),
            jax.ShapeDtypeStruct((_H, _L), jnp.int32),
            jax.ShapeDtypeStruct((_H, _L), jnp.int32),
        ],
        scratch_shapes=[
            pltpu.VMEM((_T, _L), jnp.float32),
            pltpu.VMEM((_H, _L), jnp.float32),
            pltpu.VMEM((_H, _L), jnp.float32),
            pltpu.VMEM((_L, _H), jnp.float32),
            pltpu.VMEM((_L * _L, _MT), jnp.float32),
            pltpu.VMEM((_L, _H), jnp.float32),
            pltpu.VMEM((_L, _H), jnp.int32),
            pltpu.VMEM((_L, _H), jnp.int32),
        ],
    )(
        feats,
        W,
        b.reshape(1, _L),
        start_transition.reshape(1, _L),
        transitions,
        transitions.T,
        end_transition.reshape(_L, 1),
    )
    bps_full = jnp.stack([bpsa, bpsb], axis=1).reshape(_T * _L)
    path = _sc_backtrace(bps_full, last16.reshape(16))
    return score[0, 0], path
